# baseline (speedup 1.0000x reference)
import functools

import jax
import jax.numpy as jnp
from jax import lax
from jax.experimental import pallas as pl
from jax.experimental.pallas import tpu as pltpu
from jax.experimental.pallas import tpu_sc as plsc

B = 16384
D = 16
H = 512
NT = 5
NF = 8
E_COLS = 128

NC = 2
NS = 16
NW = NC * NS
BPW = B // NW
CHUNK = 128
NCH = BPW // CHUNK

_MESH = dict(core_axis_name="c", subcore_axis_name="s",
             num_cores=NC, num_subcores=NS)


def _make_sc_gather(table_ids):
  nj = len(table_ids)
  mesh = plsc.VectorSubcoreMesh(**_MESH)

  @functools.partial(
      pl.kernel,
      out_type=jax.ShapeDtypeStruct((B, E_COLS), jnp.float32),
      mesh=mesh,
      compiler_params=pltpu.CompilerParams(use_tc_tiling_on_sc=False),
      scratch_types=[
          pltpu.VMEM((nj, NCH, CHUNK), jnp.int32),
          pltpu.VMEM((nj, BPW, D), jnp.float32),
          pltpu.SemaphoreType.DMA,
      ],
  )
  def gather_kernel(idx_hbm, *rest):
    tabs = rest[:nj]
    out_hbm = rest[nj]
    idx_v, rows_v, sem = rest[nj + 1:]
    wid = lax.axis_index("s") * NC + lax.axis_index("c")
    base = wid * BPW
    for k, j in enumerate(table_ids):
      pltpu.sync_copy(idx_hbm.at[j, pl.ds(wid * NCH, NCH)], idx_v.at[k])
    copies = []
    for k in range(nj):
      for c in range(NCH):
        copies.append(pltpu.async_copy(
            tabs[k].at[idx_v.at[k, c]],
            rows_v.at[k, pl.ds(c * CHUNK, CHUNK)],
            sem))
    for cp in copies:
      cp.wait()
    for k, j in enumerate(table_ids):
      pltpu.sync_copy(rows_v.at[k],
                      out_hbm.at[pl.ds(base, BPW), pl.ds(j * D, D)])

  return gather_kernel


MB = 1024


def _tc_dense_kernel(ea_ref, eb_ref, x_ref, we_ref, wx_ref, b_ref, out_ref):
  we = we_ref[...]
  acc = jnp.dot(eb_ref[:, :D].astype(jnp.bfloat16), we[:D],
                preferred_element_type=jnp.float32)
  acc += jnp.dot(ea_ref[:, D:NT * D].astype(jnp.bfloat16), we[D:],
                 preferred_element_type=jnp.float32)
  acc += jnp.dot(x_ref[...], wx_ref[...], preferred_element_type=jnp.float32)
  out_ref[...] = acc + b_ref[...]


def _tc_dense(ea, eb, x, we, wx, b2):
  return pl.pallas_call(
      _tc_dense_kernel,
      grid=(B // MB,),
      in_specs=[
          pl.BlockSpec((MB, E_COLS), lambda i: (i, 0)),
          pl.BlockSpec((MB, E_COLS), lambda i: (i, 0)),
          pl.BlockSpec((MB, NF), lambda i: (i, 0)),
          pl.BlockSpec((NT * D, H), lambda i: (0, 0)),
          pl.BlockSpec((NF, H), lambda i: (0, 0)),
          pl.BlockSpec((1, H), lambda i: (0, 0)),
      ],
      out_specs=pl.BlockSpec((MB, H), lambda i: (i, 0)),
      out_shape=jax.ShapeDtypeStruct((B, H), jnp.float32),
  )(ea, eb, x, we, wx, b2)


def kernel(x, emb_sid, emb_scat, emb_eid, emb_ecat, emb_len, W, b):
  idx = x[:, 3:3 + NT].astype(jnp.int32).T.reshape(NT, NW * NCH, CHUNK)
  ea = _make_sc_gather((1, 2, 3, 4))(idx, emb_scat, emb_eid, emb_ecat,
                                     emb_len)
  eb = _make_sc_gather((0,))(idx, emb_sid)
  we = W[:NT * D].astype(jnp.bfloat16)
  wx = jnp.zeros((NF, H), jnp.float32).at[0:3].set(W[NT * D:])
  b2 = b.reshape(1, H)
  return _tc_dense(ea, eb, x, we, wx, b2)


# speedup vs baseline: 1.0381x; 1.0381x over previous
import functools

import jax
import jax.numpy as jnp
from jax import lax
from jax.experimental import pallas as pl
from jax.experimental.pallas import tpu as pltpu
from jax.experimental.pallas import tpu_sc as plsc

B = 16384
D = 16
H = 512
NT = 5
NF = 8
E_COLS = 128

NC = 2
NS = 16
NW = NC * NS
BPW = B // NW
CHUNK = 128
NCH = BPW // CHUNK


def _sc_gather(idx, t_sid, t_scat, t_eid, t_ecat, t_len):
  mesh = plsc.VectorSubcoreMesh(
      core_axis_name="c", subcore_axis_name="s",
      num_cores=NC, num_subcores=NS)

  @functools.partial(
      pl.kernel,
      out_type=jax.ShapeDtypeStruct((B, E_COLS), jnp.float32),
      mesh=mesh,
      compiler_params=pltpu.CompilerParams(use_tc_tiling_on_sc=False),
      scratch_types=[
          pltpu.VMEM((NT, NCH, CHUNK), jnp.int32),
          pltpu.VMEM((NT, BPW, D), jnp.float32),
          pltpu.SemaphoreType.DMA,
      ],
  )
  def gather_kernel(idx_hbm, tab0, tab1, tab2, tab3, tab4, out_hbm,
                    idx_v, rows_v, sem):
    tabs = [tab0, tab1, tab2, tab3, tab4]
    wid = lax.axis_index("s") * NC + lax.axis_index("c")
    base = wid * BPW
    for j in range(NT):
      pltpu.sync_copy(idx_hbm.at[j, pl.ds(wid * NCH, NCH)], idx_v.at[j])
    copies = []
    for j in range(NT):
      for c in range(NCH):
        copies.append(pltpu.async_copy(
            tabs[j].at[idx_v.at[j, c]],
            rows_v.at[j, pl.ds(c * CHUNK, CHUNK)],
            sem))
    for cp in copies:
      cp.wait()
    for j in range(NT):
      pltpu.sync_copy(rows_v.at[j],
                      out_hbm.at[pl.ds(base, BPW), pl.ds(j * D, D)])

  return gather_kernel(idx, t_sid, t_scat, t_eid, t_ecat, t_len)


MB = 1024


def _tc_dense_kernel(e_ref, xt_ref, we_ref, wx_ref, b_ref, out_ref):
  acc = jnp.dot(e_ref[:, :NT * D].astype(jnp.bfloat16), we_ref[...],
                preferred_element_type=jnp.float32)
  acc += lax.dot_general(xt_ref[...], wx_ref[...],
                         (((0,), (0,)), ((), ())),
                         preferred_element_type=jnp.float32)
  out_ref[...] = acc + b_ref[...]


def _tc_dense(e, xt, we, wx, b2):
  return pl.pallas_call(
      _tc_dense_kernel,
      grid=(B // MB,),
      in_specs=[
          pl.BlockSpec((MB, E_COLS), lambda i: (i, 0)),
          pl.BlockSpec((NF, MB), lambda i: (0, i)),
          pl.BlockSpec((NT * D, H), lambda i: (0, 0)),
          pl.BlockSpec((NF, H), lambda i: (0, 0)),
          pl.BlockSpec((1, H), lambda i: (0, 0)),
      ],
      out_specs=pl.BlockSpec((MB, H), lambda i: (i, 0)),
      out_shape=jax.ShapeDtypeStruct((B, H), jnp.float32),
  )(e, xt, we, wx, b2)


def kernel(x, emb_sid, emb_scat, emb_eid, emb_ecat, emb_len, W, b):
  idx = x[:, 3:3 + NT].astype(jnp.int32).T.reshape(NT, NW * NCH, CHUNK)
  e = _sc_gather(idx, emb_sid, emb_scat, emb_eid, emb_ecat, emb_len)
  we = W[:NT * D].astype(jnp.bfloat16)
  wx = jnp.zeros((NF, H), jnp.float32).at[0:3].set(W[NT * D:])
  b2 = b.reshape(1, H)
  return _tc_dense(e, x.T, we, wx, b2)
